# baseline (speedup 1.0000x reference)
import functools

import numpy as np
import jax
import jax.numpy as jnp
from jax import lax
from jax.experimental import pallas as pl
from jax.experimental.pallas import tpu as pltpu
from jax.experimental.pallas import tpu_sc as plsc

MAX_SEQ = 4096
D_MODEL = 1024
BATCH = 4
SEQ = 4096

_INFO = plsc.get_sparse_core_info()
NC, NS, L = _INFO.num_cores, _INFO.num_subcores, _INFO.num_lanes
NW = NC * NS
N_ROWS = BATCH * SEQ
ROWS_PER_W = N_ROWS // NW
CHUNK = 32
NCHUNK = ROWS_PER_W // CHUNK


def _make_pe_np() -> np.ndarray:
    pos = np.arange(MAX_SEQ, dtype=np.float32)[:, None]
    i = np.arange(D_MODEL, dtype=np.float32)[None, :]
    angles = pos / np.power(10000.0, 2.0 * np.floor(i / 2.0) / D_MODEL)
    even = (np.arange(D_MODEL) % 2 == 0)[None, :]
    pe = np.where(even, np.sin(angles), np.cos(angles))
    return pe.astype(np.float32)


_PE_NP = _make_pe_np()

_mesh = plsc.VectorSubcoreMesh(core_axis_name="c", subcore_axis_name="s")


@functools.partial(
    pl.kernel,
    out_type=jax.ShapeDtypeStruct((N_ROWS, D_MODEL), jnp.float32),
    mesh=_mesh,
    scratch_types=[
        pltpu.VMEM((NCHUNK, CHUNK), jnp.int32),
        pltpu.VMEM((CHUNK, D_MODEL), jnp.float32),
        pltpu.VMEM((CHUNK, D_MODEL), jnp.float32),
        pltpu.SemaphoreType.DMA,
    ],
)
def _embed_sc(table_hbm, idx_hbm, pe_hbm, out_hbm, idx_v, pe_v, rows_v, sem):
    wid = lax.axis_index("s") * NC + lax.axis_index("c")
    base = wid * ROWS_PER_W
    pos0 = base % SEQ
    pltpu.sync_copy(idx_hbm.at[wid], idx_v)

    def chunk_body(c, _):
        pltpu.sync_copy(pe_hbm.at[pl.ds(pos0 + c * CHUNK, CHUNK)], pe_v)
        pltpu.async_copy(table_hbm.at[idx_v.at[c]], rows_v, sem).wait()

        def add_row(i, _):
            for j in range(D_MODEL // L):
                sl = pl.ds(j * L, L)
                rows_v[i, sl] += pe_v[i, sl]
            return 0

        lax.fori_loop(0, CHUNK, add_row, 0)
        pltpu.sync_copy(rows_v, out_hbm.at[pl.ds(base + c * CHUNK, CHUNK)])
        return 0

    lax.fori_loop(0, NCHUNK, chunk_body, 0)


def kernel(x, token_table):
    idx = x.astype(jnp.int32).reshape(NW, NCHUNK, CHUNK)
    pe = jnp.asarray(_PE_NP)
    out = _embed_sc(token_table, idx, pe)
    return out.reshape(BATCH, SEQ, D_MODEL)


# speedup vs baseline: 1.0897x; 1.0897x over previous
import functools

import numpy as np
import jax
import jax.numpy as jnp
from jax import lax
from jax.experimental import pallas as pl
from jax.experimental.pallas import tpu as pltpu
from jax.experimental.pallas import tpu_sc as plsc

MAX_SEQ = 4096
D_MODEL = 1024
BATCH = 4
SEQ = 4096

_INFO = plsc.get_sparse_core_info()
NC, NS, L = _INFO.num_cores, _INFO.num_subcores, _INFO.num_lanes
NW = NC * NS
POS_PER_W = SEQ // NW
CHUNK = 16
NCH = POS_PER_W // CHUNK
NUNIT = NCH * BATCH
NSLOT = 4

_COLG = 256
_KITER = CHUNK * (D_MODEL // _COLG)


def _make_pe_np() -> np.ndarray:
    pos = np.arange(MAX_SEQ, dtype=np.float32)[:, None]
    i = np.arange(D_MODEL, dtype=np.float32)[None, :]
    angles = pos / np.power(10000.0, 2.0 * np.floor(i / 2.0) / D_MODEL)
    even = (np.arange(D_MODEL) % 2 == 0)[None, :]
    pe = np.where(even, np.sin(angles), np.cos(angles))
    return pe.astype(np.float32)


_PE_NP = _make_pe_np()

_mesh = plsc.VectorSubcoreMesh(core_axis_name="c", subcore_axis_name="s")


@functools.partial(
    pl.kernel,
    out_type=jax.ShapeDtypeStruct((BATCH * SEQ, D_MODEL), jnp.float32),
    mesh=_mesh,
    scratch_types=[
        pltpu.VMEM((NUNIT, CHUNK), jnp.int32),
        pltpu.VMEM((2, CHUNK, D_MODEL), jnp.float32),
        pltpu.VMEM((NSLOT, CHUNK, D_MODEL), jnp.float32),
        pltpu.SemaphoreType.DMA,
        pltpu.SemaphoreType.DMA,
        pltpu.SemaphoreType.DMA,
    ],
)
def _embed_sc(table_hbm, idx_hbm, pe_hbm, out_hbm, idx_v, pe_v, rows_v,
              gsem, osem, psem):
    wid = lax.axis_index("s") * NC + lax.axis_index("c")
    pos0 = wid * POS_PER_W

    pltpu.sync_copy(idx_hbm.at[wid], idx_v)

    def gather(u):
        return pltpu.async_copy(
            table_hbm.at[idx_v.at[u]], rows_v.at[u % NSLOT], gsem)

    def pe_fetch(c):
        return pltpu.async_copy(
            pe_hbm.at[pl.ds(pos0 + c * CHUNK, CHUNK)], pe_v.at[c % 2], psem)

    pe_pend = pe_fetch(0)
    g_pend = [gather(0), gather(1)]
    o_pend = []

    for u in range(NUNIT):
        slot = u % NSLOT
        c, b = divmod(u, BATCH)

        if u + 2 < NUNIT:
            if u >= 2:
                o_pend.pop(0).wait()
            g_pend.append(gather(u + 2))

        if b == 0:
            pe_pend.wait()
            if c + 1 < NCH:
                pe_pend = pe_fetch(c + 1)

        g_pend.pop(0).wait()

        pec = c % 2

        def add_body(k, _, slot=slot, pec=pec):
            i = lax.shift_right_logical(k, 2)
            g = (k & 3) * _COLG
            for j in range(_COLG // L):
                sl = pl.ds(g + j * L, L)
                rows_v[slot, i, sl] += pe_v[pec, i, sl]
            return 0

        lax.fori_loop(0, _KITER, add_body, 0)

        orow = b * SEQ + pos0 + c * CHUNK
        o_pend.append(pltpu.async_copy(
            rows_v.at[slot], out_hbm.at[pl.ds(orow, CHUNK)], osem))

    for o in o_pend:
        o.wait()


def kernel(x, token_table):
    idx = (x.astype(jnp.int32)
           .reshape(BATCH, NW, NCH, CHUNK)
           .transpose(1, 2, 0, 3)
           .reshape(NW, NUNIT, CHUNK))
    pe = jnp.asarray(_PE_NP)
    out = _embed_sc(token_table, idx, pe)
    return out.reshape(BATCH, SEQ, D_MODEL)


# speedup vs baseline: 1.3122x; 1.2042x over previous
import functools

import numpy as np
import jax
import jax.numpy as jnp
from jax import lax
from jax.experimental import pallas as pl
from jax.experimental.pallas import tpu as pltpu
from jax.experimental.pallas import tpu_sc as plsc

MAX_SEQ = 4096
D_MODEL = 1024
BATCH = 4
SEQ = 4096

_INFO = plsc.get_sparse_core_info()
NC, NS, L = _INFO.num_cores, _INFO.num_subcores, _INFO.num_lanes
NW = NC * NS
POS_PER_W = SEQ // NW
CHUNK = 16
NCH = POS_PER_W // CHUNK
NUNIT = NCH * BATCH
NSLOT = 4

_COLG = 256
_KITER = CHUNK * (D_MODEL // _COLG)


def _make_pe_np() -> np.ndarray:
    pos = np.arange(MAX_SEQ, dtype=np.float32)[:, None]
    i = np.arange(D_MODEL, dtype=np.float32)[None, :]
    angles = pos / np.power(10000.0, 2.0 * np.floor(i / 2.0) / D_MODEL)
    even = (np.arange(D_MODEL) % 2 == 0)[None, :]
    pe = np.where(even, np.sin(angles), np.cos(angles))
    return pe.astype(np.float32)


_PE_NP = _make_pe_np()

_mesh = plsc.VectorSubcoreMesh(core_axis_name="c", subcore_axis_name="s")


@functools.partial(
    pl.kernel,
    out_type=jax.ShapeDtypeStruct((BATCH * SEQ, D_MODEL), jnp.float32),
    mesh=_mesh,
    scratch_types=[
        pltpu.VMEM((NUNIT, CHUNK), jnp.int32),
        pltpu.VMEM((2, CHUNK, D_MODEL), jnp.float32),
        pltpu.VMEM((NSLOT, CHUNK, D_MODEL), jnp.float32),
        pltpu.SemaphoreType.DMA,
        pltpu.SemaphoreType.DMA,
        pltpu.SemaphoreType.DMA,
    ],
)
def _embed_sc(table_hbm, idx_hbm, pe_hbm, out_hbm, idx_v, pe_v, rows_v,
              gsem, osem, psem):
    wid = lax.axis_index("s") * NC + lax.axis_index("c")
    pos0 = wid * POS_PER_W

    pltpu.sync_copy(idx_hbm.at[wid], idx_v)

    def gather(u):
        return pltpu.async_copy(
            table_hbm.at[idx_v.at[u]], rows_v.at[u % NSLOT], gsem)

    def pe_fetch(c):
        return pltpu.async_copy(
            pe_hbm.at[pl.ds(pos0 + c * CHUNK, CHUNK)], pe_v.at[c % 2], psem)

    pe_pend = pe_fetch(0)
    g_pend = [gather(0), gather(1)]
    o_pend = []

    for u in range(NUNIT):
        slot = u % NSLOT
        c, b = divmod(u, BATCH)

        if u + 2 < NUNIT:
            if u >= 2:
                o_pend.pop(0).wait()
            g_pend.append(gather(u + 2))

        if b == 0:
            pe_pend.wait()
            if c + 1 < NCH:
                pe_pend = pe_fetch(c + 1)

        g_pend.pop(0).wait()

        pec = c % 2

        def add_body(k, _, slot=slot, pec=pec):
            i = lax.shift_right_logical(k, 2)
            g = (k & 3) * _COLG
            for j in range(_COLG // L):
                sl = pl.ds(g + j * L, L)
                plsc.addupdate(rows_v.at[slot, i, sl], pe_v[pec, i, sl])
            return 0

        lax.fori_loop(0, _KITER, add_body, 0)

        orow = b * SEQ + pos0 + c * CHUNK
        o_pend.append(pltpu.async_copy(
            rows_v.at[slot], out_hbm.at[pl.ds(orow, CHUNK)], osem))

    for o in o_pend:
        o.wait()


def kernel(x, token_table):
    idx = (x.astype(jnp.int32)
           .reshape(BATCH, NW, NCH, CHUNK)
           .transpose(1, 2, 0, 3)
           .reshape(NW, NUNIT, CHUNK))
    pe = jnp.asarray(_PE_NP)
    out = _embed_sc(token_table, idx, pe)
    return out.reshape(BATCH, SEQ, D_MODEL)


# speedup vs baseline: 2.0327x; 1.5490x over previous
import functools

import numpy as np
import jax
import jax.numpy as jnp
from jax import lax
from jax.experimental import pallas as pl
from jax.experimental.pallas import tpu as pltpu
from jax.experimental.pallas import tpu_sc as plsc

MAX_SEQ = 4096
D_MODEL = 1024
BATCH = 4
SEQ = 4096

_INFO = plsc.get_sparse_core_info()
NC, NS, L = _INFO.num_cores, _INFO.num_subcores, _INFO.num_lanes
NW = NC * NS
POS_PER_W = SEQ // NW
CHUNK = 16
NCH = POS_PER_W // CHUNK
NUNIT = NCH * BATCH
NSLOT = 4


def _make_pe_np() -> np.ndarray:
    pos = np.arange(MAX_SEQ, dtype=np.float32)[:, None]
    i = np.arange(D_MODEL, dtype=np.float32)[None, :]
    angles = pos / np.power(10000.0, 2.0 * np.floor(i / 2.0) / D_MODEL)
    even = (np.arange(D_MODEL) % 2 == 0)[None, :]
    pe = np.where(even, np.sin(angles), np.cos(angles))
    return pe.astype(np.float32)


_PE_NP = _make_pe_np()

_mesh = plsc.VectorSubcoreMesh(core_axis_name="c", subcore_axis_name="s")


@functools.partial(
    pl.kernel,
    out_type=jax.ShapeDtypeStruct((BATCH * SEQ, D_MODEL), jnp.float32),
    mesh=_mesh,
    scratch_types=[
        pltpu.VMEM((NUNIT, CHUNK), jnp.int32),
        pltpu.VMEM((2, CHUNK, D_MODEL), jnp.float32),
        pltpu.VMEM((NSLOT, CHUNK, D_MODEL), jnp.float32),
        pltpu.SemaphoreType.DMA((NSLOT,)),
        pltpu.SemaphoreType.DMA,
    ],
)
def _embed_sc(table_hbm, idx_hbm, pe_hbm, out_hbm, idx_v, pe_v, rows_v,
              sems, psem):
    wid = lax.axis_index("s") * NC + lax.axis_index("c")
    pos0 = wid * POS_PER_W

    pltpu.sync_copy(idx_hbm.at[wid], idx_v)

    def issue_gather(u):
        pltpu.async_copy(
            table_hbm.at[idx_v.at[u]], rows_v.at[u & 3], sems.at[u & 3])

    def issue_pe(c):
        pltpu.async_copy(
            pe_hbm.at[pl.ds(pos0 + c * CHUNK, CHUNK)], pe_v.at[c & 1], psem)

    def wait_out(slot):
        pltpu.make_async_copy(
            rows_v.at[0], out_hbm.at[pl.ds(0, CHUNK)], sems.at[slot]).wait()

    def wait_gather(slot):
        pltpu.make_async_copy(
            table_hbm.at[idx_v.at[0]], rows_v.at[0], sems.at[slot]).wait()

    def wait_pe():
        pltpu.make_async_copy(
            pe_hbm.at[pl.ds(0, CHUNK)], pe_v.at[0], psem).wait()

    issue_pe(0)
    issue_gather(0)
    issue_gather(1)

    def unit_body(u, _):
        slot = u & 3
        c = lax.shift_right_logical(u, 2)
        s2 = (u + 2) & 3

        @pl.when(u >= 2)
        def _():
            wait_out(s2)

        @pl.when(u + 2 < NUNIT)
        def _():
            issue_gather(u + 2)

        @pl.when(slot == 0)
        def _():
            wait_pe()

            @pl.when(c + 1 < NCH)
            def _():
                issue_pe(c + 1)

        wait_gather(slot)

        pec = c & 1

        @plsc.parallel_loop(0, CHUNK)
        def add_body(i):
            for j in range(D_MODEL // L):
                sl = pl.ds(j * L, L)
                plsc.addupdate(rows_v.at[slot, i, sl], pe_v[pec, i, sl])

        orow = slot * SEQ + pos0 + c * CHUNK
        pltpu.async_copy(
            rows_v.at[slot], out_hbm.at[pl.ds(orow, CHUNK)], sems.at[slot])
        return 0

    lax.fori_loop(0, NUNIT, unit_body, 0)
    wait_out(2)
    wait_out(3)


def kernel(x, token_table):
    idx = (x.astype(jnp.int32)
           .reshape(BATCH, NW, NCH, CHUNK)
           .transpose(1, 2, 0, 3)
           .reshape(NW, NUNIT, CHUNK))
    pe = jnp.asarray(_PE_NP)
    out = _embed_sc(token_table, idx, pe)
    return out.reshape(BATCH, SEQ, D_MODEL)


# speedup vs baseline: 2.0600x; 1.0134x over previous
import functools

import numpy as np
import jax
import jax.numpy as jnp
from jax import lax
from jax.experimental import pallas as pl
from jax.experimental.pallas import tpu as pltpu
from jax.experimental.pallas import tpu_sc as plsc

MAX_SEQ = 4096
D_MODEL = 1024
BATCH = 4
SEQ = 4096

_INFO = plsc.get_sparse_core_info()
NC, NS, L = _INFO.num_cores, _INFO.num_subcores, _INFO.num_lanes
NW = NC * NS
POS_PER_W = SEQ // NW
CHUNK = 16
NCH = POS_PER_W // CHUNK
NUNIT = NCH * BATCH
NSLOT = 4


def _make_pe_np() -> np.ndarray:
    pos = np.arange(MAX_SEQ, dtype=np.float32)[:, None]
    i = np.arange(D_MODEL, dtype=np.float32)[None, :]
    angles = pos / np.power(10000.0, 2.0 * np.floor(i / 2.0) / D_MODEL)
    even = (np.arange(D_MODEL) % 2 == 0)[None, :]
    pe = np.where(even, np.sin(angles), np.cos(angles))
    return pe.astype(np.float32)


_PE_NP = _make_pe_np()

_mesh = plsc.VectorSubcoreMesh(core_axis_name="c", subcore_axis_name="s")


@functools.partial(
    pl.kernel,
    out_type=jax.ShapeDtypeStruct((BATCH * SEQ, D_MODEL), jnp.float32),
    mesh=_mesh,
    scratch_types=[
        pltpu.VMEM((BATCH, POS_PER_W), jnp.int32),
        pltpu.VMEM((2, CHUNK, D_MODEL), jnp.float32),
        pltpu.VMEM((NSLOT, CHUNK, D_MODEL), jnp.float32),
        pltpu.SemaphoreType.DMA((NSLOT,)),
        pltpu.SemaphoreType.DMA,
    ],
)
def _embed_sc(table_hbm, idx_hbm, pe_hbm, out_hbm, idx_v, pe_v, rows_v,
              sems, psem):
    wid = lax.axis_index("s") * NC + lax.axis_index("c")
    pos0 = wid * POS_PER_W

    for b in range(BATCH):
        pltpu.sync_copy(idx_hbm.at[b, pl.ds(pos0, POS_PER_W)], idx_v.at[b])

    def issue_gather(u):
        b = u & 3
        c = lax.shift_right_logical(u, 2)
        pltpu.async_copy(
            table_hbm.at[idx_v.at[b, pl.ds(c * CHUNK, CHUNK)]],
            rows_v.at[u & 3], sems.at[u & 3])

    def issue_pe(c):
        pltpu.async_copy(
            pe_hbm.at[pl.ds(pos0 + c * CHUNK, CHUNK)], pe_v.at[c & 1], psem)

    def wait_out(slot):
        pltpu.make_async_copy(
            rows_v.at[0], out_hbm.at[pl.ds(0, CHUNK)], sems.at[slot]).wait()

    def wait_gather(slot):
        pltpu.make_async_copy(
            table_hbm.at[idx_v.at[0, pl.ds(0, CHUNK)]], rows_v.at[0],
            sems.at[slot]).wait()

    def wait_pe():
        pltpu.make_async_copy(
            pe_hbm.at[pl.ds(0, CHUNK)], pe_v.at[0], psem).wait()

    issue_pe(0)
    issue_gather(0)
    issue_gather(1)

    def unit_body(u, _):
        slot = u & 3
        c = lax.shift_right_logical(u, 2)
        s2 = (u + 2) & 3

        @pl.when(u >= 2)
        def _():
            wait_out(s2)

        @pl.when(u + 2 < NUNIT)
        def _():
            issue_gather(u + 2)

        @pl.when(slot == 0)
        def _():
            wait_pe()

            @pl.when(c + 1 < NCH)
            def _():
                issue_pe(c + 1)

        wait_gather(slot)

        pec = c & 1

        @plsc.parallel_loop(0, CHUNK)
        def add_body(i):
            for j in range(D_MODEL // L):
                sl = pl.ds(j * L, L)
                plsc.addupdate(rows_v.at[slot, i, sl], pe_v[pec, i, sl])

        orow = slot * SEQ + pos0 + c * CHUNK
        pltpu.async_copy(
            rows_v.at[slot], out_hbm.at[pl.ds(orow, CHUNK)], sems.at[slot])
        return 0

    lax.fori_loop(0, NUNIT, unit_body, 0)
    wait_out(2)
    wait_out(3)


def kernel(x, token_table):
    pe = jnp.asarray(_PE_NP)
    out = _embed_sc(token_table, x.astype(jnp.int32), pe)
    return out.reshape(BATCH, SEQ, D_MODEL)
